# Initial kernel scaffold; baseline (speedup 1.0000x reference)
import jax
import jax.numpy as jnp
from jax.experimental import pallas as pl


def kernel(x, edge_index, W1, b1, W2, b2, Wlp, blp):
    raise NotImplementedError("write your pallas kernel here")


# speedup vs baseline: 14.6282x; 14.6282x over previous
import functools

import jax
import jax.numpy as jnp
from jax import lax
from jax.experimental import pallas as pl
from jax.experimental.pallas import tpu as pltpu
from jax.experimental.pallas import tpu_sc as plsc

_N = 646
_E = 20672
_NCOL = 648
_NP = 648
_S2 = _N * _NCOL
_DUMMY_ROW = _N

_NWORK = 32
_PER_W = 768
_EP = _NWORK * _PER_W
_CHUNK = 128
_NCHUNK = _PER_W // _CHUNK
_Z = 26176
_SP_ALLOC = 16 * _Z


def _sc_scatter_body(edges_hbm, out_hbm, src_v, dst_v, idx_v, ones_v,
                     zbuf_v, shared):
    c = lax.axis_index("c")
    s = lax.axis_index("s")
    wid = c * 16 + s
    base = wid * _PER_W
    pltpu.sync_copy(edges_hbm.at[pl.ds(base, _PER_W)], src_v)
    pltpu.sync_copy(edges_hbm.at[pl.ds(_EP + base, _PER_W)], dst_v)

    ones16 = jnp.ones((16,), jnp.float32)
    zeros16 = jnp.zeros((16,), jnp.float32)
    for t in range(_CHUNK // 16):
        ones_v[pl.ds(t * 16, 16)] = ones16

    def _zero_body(i, carry):
        zbuf_v[pl.ds(i * 16, 16)] = zeros16
        return carry

    lax.fori_loop(0, _Z // 16, _zero_body, 0)

    for t in range(_PER_W // 16):
        sv = src_v[pl.ds(t * 16, 16)]
        dv = dst_v[pl.ds(t * 16, 16)]
        idx_v[t // 8, pl.ds((t % 8) * 16, 16)] = dv * _NCOL + sv

    pltpu.sync_copy(zbuf_v, shared.at[pl.ds(s * _Z, _Z)])
    plsc.subcore_barrier()
    for j in range(_NCHUNK):
        pltpu.sync_copy(ones_v, shared.at[idx_v.at[j]], add=True)
    plsc.subcore_barrier()
    pltpu.sync_copy(shared.at[pl.ds(s * _Z, _Z)], zbuf_v)
    pltpu.sync_copy(zbuf_v, out_hbm.at[pl.ds(c * _SP_ALLOC + s * _Z, _Z)])


def _make_sc_scatter():
    return pl.kernel(
        _sc_scatter_body,
        out_type=jax.ShapeDtypeStruct((2 * _SP_ALLOC,), jnp.float32),
        mesh=plsc.VectorSubcoreMesh(core_axis_name="c", subcore_axis_name="s"),
        scratch_types=[
            pltpu.VMEM((_PER_W,), jnp.int32),
            pltpu.VMEM((_PER_W,), jnp.int32),
            pltpu.VMEM((_NCHUNK, _CHUNK), jnp.int32),
            pltpu.VMEM((_CHUNK,), jnp.float32),
            pltpu.VMEM((_Z,), jnp.float32),
            pltpu.VMEM_SHARED((_SP_ALLOC,), jnp.float32),
        ],
    )


def _tc_body(p_ref, x_ref, w1_ref, b1_ref, w2_ref, b2_ref, wlp_ref, blp_ref,
             out_ref):
    bmat = p_ref[0] + p_ref[1]
    rows = lax.broadcasted_iota(jnp.int32, (_NP, _NCOL), 0)
    cols = lax.broadcasted_iota(jnp.int32, (_NP, _NCOL), 1)
    eye = jnp.where((rows == cols) & (rows < _N), 1.0, 0.0)
    bmat = bmat + eye
    deg = jnp.sum(bmat, axis=1, keepdims=True)
    dinv = lax.rsqrt(jnp.maximum(deg, 1e-12))

    x = x_ref[...]
    h1 = jnp.dot(x, w1_ref[...], preferred_element_type=jnp.float32)
    agg1 = dinv * jnp.dot(bmat, dinv * h1,
                          preferred_element_type=jnp.float32) + b1_ref[...]
    res = jnp.dot(x, wlp_ref[...], preferred_element_type=jnp.float32)
    h = res + blp_ref[...] + agg1
    h = h * (1.0 / (1.0 + jnp.exp(-h)))

    rmask = lax.broadcasted_iota(jnp.int32, (_NP, 64), 0) < _N
    cnt = float(_N * 64)
    mu = jnp.sum(jnp.where(rmask, h, 0.0)) / cnt
    dev = jnp.where(rmask, h - mu, 0.0)
    var = jnp.sum(dev * dev) / cnt
    hn = (h - mu) * lax.rsqrt(var + 1e-5)

    h2 = jnp.dot(hn, w2_ref[...], preferred_element_type=jnp.float32)
    out_ref[...] = dinv * jnp.dot(bmat, dinv * h2,
                                  preferred_element_type=jnp.float32) + b2_ref[...]


def kernel(x, edge_index, W1, b1, W2, b2, Wlp, blp):
    ei = edge_index.astype(jnp.int32)
    pad = _EP - _E
    src_p = jnp.concatenate([ei[0], jnp.zeros((pad,), jnp.int32)])
    dst_p = jnp.concatenate([ei[1], jnp.full((pad,), _DUMMY_ROW, jnp.int32)])
    edges = jnp.concatenate([src_p, dst_p])

    parts = _make_sc_scatter()(edges).reshape(2, _SP_ALLOC)
    p = parts[:, :_S2].reshape(2, _N, _NCOL)
    p = jnp.pad(p, ((0, 0), (0, _NP - _N), (0, 0)))

    xp = jnp.pad(x, ((0, _NP - _N), (0, 0)))
    out = pl.pallas_call(
        _tc_body,
        out_shape=jax.ShapeDtypeStruct((_NP, 64), jnp.float32),
    )(p, xp, W1, b1.reshape(1, -1), W2, b2.reshape(1, -1),
      Wlp, blp.reshape(1, -1))
    return out[:_N]


# speedup vs baseline: 15.2734x; 1.0441x over previous
import jax
import jax.numpy as jnp
from jax import lax
from jax.experimental import pallas as pl
from jax.experimental.pallas import tpu as pltpu
from jax.experimental.pallas import tpu_sc as plsc

_N = 646
_E = 20672
_NCOL = 648
_NP = 648
_NROW = 704
_PAD_DST = _N

_NTILE = 32
_EP = 24576
_ROWS_T = _NROW // _NTILE
_Z = _ROWS_T * _NCOL


def _sc_scatter_body(edges_hbm, out_hbm, src_v, dst_v, bacc_v):
    c = lax.axis_index("c")
    s = lax.axis_index("s")
    w = c * 16 + s
    pltpu.sync_copy(edges_hbm.at[pl.ds(0, _EP)], src_v)
    pltpu.sync_copy(edges_hbm.at[pl.ds(_EP, _EP)], dst_v)

    ones16 = jnp.ones((16,), jnp.float32)
    zeros16 = jnp.zeros((16,), jnp.float32)

    def _zero_body(i, carry):
        for u in range(8):
            bacc_v[pl.ds((i * 8 + u) * 16, 16)] = zeros16
        return carry

    nz = _Z // 16
    lax.fori_loop(0, nz // 8, _zero_body, 0)
    for u in range(nz - (nz // 8) * 8):
        bacc_v[pl.ds(((nz // 8) * 8 + u) * 16, 16)] = zeros16

    base = w * _ROWS_T

    def _scatter_body(i, carry):
        for u in range(4):
            t = i * 4 + u
            sv = src_v[pl.ds(t * 16, 16)]
            dv = dst_v[pl.ds(t * 16, 16)] - base
            loc = dv * _NCOL + sv
            valid = (loc >= 0) & (loc < _Z)
            plsc.addupdate_scatter(bacc_v, [loc], ones16, mask=valid)
        return carry

    lax.fori_loop(0, _EP // 64, _scatter_body, 0)
    pltpu.sync_copy(bacc_v, out_hbm.at[pl.ds(w * _Z, _Z)])


def _make_sc_scatter():
    return pl.kernel(
        _sc_scatter_body,
        out_type=jax.ShapeDtypeStruct((_NROW * _NCOL,), jnp.float32),
        mesh=plsc.VectorSubcoreMesh(core_axis_name="c", subcore_axis_name="s"),
        compiler_params=pltpu.CompilerParams(needs_layout_passes=False),
        scratch_types=[
            pltpu.VMEM((_EP,), jnp.int32),
            pltpu.VMEM((_EP,), jnp.int32),
            pltpu.VMEM((_Z,), jnp.float32),
        ],
    )


def _tc_body(p_ref, x_ref, w1_ref, b1_ref, w2_ref, b2_ref, wlp_ref, blp_ref,
             out_ref):
    bmat = p_ref[pl.ds(0, _NP), :]
    rows = lax.broadcasted_iota(jnp.int32, (_NP, _NCOL), 0)
    cols = lax.broadcasted_iota(jnp.int32, (_NP, _NCOL), 1)
    eye = jnp.where((rows == cols) & (rows < _N), 1.0, 0.0)
    bmat = bmat + eye
    deg = jnp.sum(bmat, axis=1, keepdims=True)
    dinv = lax.rsqrt(jnp.maximum(deg, 1e-12))

    x = x_ref[...]
    h1 = jnp.dot(x, w1_ref[...], preferred_element_type=jnp.float32)
    agg1 = dinv * jnp.dot(bmat, dinv * h1,
                          preferred_element_type=jnp.float32) + b1_ref[...]
    res = jnp.dot(x, wlp_ref[...], preferred_element_type=jnp.float32)
    h = res + blp_ref[...] + agg1
    h = h * (1.0 / (1.0 + jnp.exp(-h)))

    rmask = lax.broadcasted_iota(jnp.int32, (_NP, 64), 0) < _N
    cnt = float(_N * 64)
    mu = jnp.sum(jnp.where(rmask, h, 0.0)) / cnt
    dev = jnp.where(rmask, h - mu, 0.0)
    var = jnp.sum(dev * dev) / cnt
    hn = (h - mu) * lax.rsqrt(var + 1e-5)

    h2 = jnp.dot(hn, w2_ref[...], preferred_element_type=jnp.float32)
    out_ref[...] = dinv * jnp.dot(bmat, dinv * h2,
                                  preferred_element_type=jnp.float32) + b2_ref[...]


def kernel(x, edge_index, W1, b1, W2, b2, Wlp, blp):
    ei = edge_index.astype(jnp.int32)
    pad = _EP - _E
    src_p = jnp.concatenate([ei[0], jnp.zeros((pad,), jnp.int32)])
    dst_p = jnp.concatenate([ei[1], jnp.full((pad,), _PAD_DST, jnp.int32)])
    edges = jnp.concatenate([src_p, dst_p])

    p = _make_sc_scatter()(edges).reshape(_NROW, _NCOL)

    xp = jnp.pad(x, ((0, _NP - _N), (0, 0)))
    out = pl.pallas_call(
        _tc_body,
        out_shape=jax.ShapeDtypeStruct((_NP, 64), jnp.float32),
    )(p, xp, W1, b1.reshape(1, -1), W2, b2.reshape(1, -1),
      Wlp, blp.reshape(1, -1))
    return out[:_N]


# speedup vs baseline: 19.2598x; 1.2610x over previous
import jax
import jax.numpy as jnp
from jax import lax
from jax.experimental import pallas as pl
from jax.experimental.pallas import tpu as pltpu
from jax.experimental.pallas import tpu_sc as plsc

_N = 646
_E = 20672
_NCOL = 648
_NP = 648
_NROW = 704
_PAD_DST = _N

_NTILE = 32
_EP = 24576
_ROWS_T = _NROW // _NTILE
_Z = _ROWS_T * _NCOL
_ZPAD = 14336


def _sc_scatter_body(edges_hbm, out_hbm, src_v, dst_v, bacc_v):
    c = lax.axis_index("c")
    s = lax.axis_index("s")
    w = c * 16 + s
    pltpu.sync_copy(edges_hbm.at[pl.ds(0, _EP)], src_v)
    pltpu.sync_copy(edges_hbm.at[pl.ds(_EP, _EP)], dst_v)

    ones16 = jnp.ones((16,), jnp.float32)
    zeros16 = jnp.zeros((16,), jnp.float32)

    @plsc.parallel_loop(0, _ZPAD // 16, unroll=8)
    def _zero_body(i):
        bacc_v[pl.ds(i * 16, 16)] = zeros16

    base = w * _ROWS_T

    @plsc.parallel_loop(0, _EP // 16, unroll=8)
    def _scatter_body(t):
        sv = src_v[pl.ds(t * 16, 16)]
        dv = dst_v[pl.ds(t * 16, 16)] - base
        loc = dv * _NCOL + sv
        valid = (loc >= 0) & (loc < _Z)
        plsc.addupdate_scatter(bacc_v, [loc], ones16, mask=valid)

    pltpu.sync_copy(bacc_v.at[pl.ds(0, _Z)], out_hbm.at[pl.ds(w * _Z, _Z)])


def _make_sc_scatter():
    return pl.kernel(
        _sc_scatter_body,
        out_type=jax.ShapeDtypeStruct((_NROW * _NCOL,), jnp.float32),
        mesh=plsc.VectorSubcoreMesh(core_axis_name="c", subcore_axis_name="s"),
        compiler_params=pltpu.CompilerParams(needs_layout_passes=False),
        scratch_types=[
            pltpu.VMEM((_EP,), jnp.int32),
            pltpu.VMEM((_EP,), jnp.int32),
            pltpu.VMEM((_ZPAD,), jnp.float32),
        ],
    )


def _tc_body(p_ref, x_ref, w1_ref, b1_ref, w2_ref, b2_ref, wlp_ref, blp_ref,
             out_ref):
    bmat = p_ref[pl.ds(0, _NP), :]
    rows = lax.broadcasted_iota(jnp.int32, (_NP, _NCOL), 0)
    cols = lax.broadcasted_iota(jnp.int32, (_NP, _NCOL), 1)
    eye = jnp.where((rows == cols) & (rows < _N), 1.0, 0.0)
    bmat = bmat + eye
    deg = jnp.sum(bmat, axis=1, keepdims=True)
    dinv = lax.rsqrt(jnp.maximum(deg, 1e-12))

    x = x_ref[...]
    h1 = jnp.dot(x, w1_ref[...], preferred_element_type=jnp.float32)
    agg1 = dinv * jnp.dot(bmat, dinv * h1,
                          preferred_element_type=jnp.float32) + b1_ref[...]
    res = jnp.dot(x, wlp_ref[...], preferred_element_type=jnp.float32)
    h = res + blp_ref[...] + agg1
    h = h * (1.0 / (1.0 + jnp.exp(-h)))

    rmask = lax.broadcasted_iota(jnp.int32, (_NP, 64), 0) < _N
    cnt = float(_N * 64)
    mu = jnp.sum(jnp.where(rmask, h, 0.0)) / cnt
    dev = jnp.where(rmask, h - mu, 0.0)
    var = jnp.sum(dev * dev) / cnt
    hn = (h - mu) * lax.rsqrt(var + 1e-5)

    h2 = jnp.dot(hn, w2_ref[...], preferred_element_type=jnp.float32)
    out_ref[...] = dinv * jnp.dot(bmat, dinv * h2,
                                  preferred_element_type=jnp.float32) + b2_ref[...]


def kernel(x, edge_index, W1, b1, W2, b2, Wlp, blp):
    ei = edge_index.astype(jnp.int32)
    pad = _EP - _E
    src_p = jnp.concatenate([ei[0], jnp.zeros((pad,), jnp.int32)])
    dst_p = jnp.concatenate([ei[1], jnp.full((pad,), _PAD_DST, jnp.int32)])
    edges = jnp.concatenate([src_p, dst_p])

    p = _make_sc_scatter()(edges).reshape(_NROW, _NCOL)

    xp = jnp.pad(x, ((0, _NP - _N), (0, 0)))
    out = pl.pallas_call(
        _tc_body,
        out_shape=jax.ShapeDtypeStruct((_NP, 64), jnp.float32),
    )(p, xp, W1, b1.reshape(1, -1), W2, b2.reshape(1, -1),
      Wlp, blp.reshape(1, -1))
    return out[:_N]


# speedup vs baseline: 19.5286x; 1.0140x over previous
import jax
import jax.numpy as jnp
from jax import lax
from jax.experimental import pallas as pl
from jax.experimental.pallas import tpu as pltpu
from jax.experimental.pallas import tpu_sc as plsc

_N = 646
_E = 20672
_NCOL = 648
_NP = 648
_NROW = 704
_PAD_DST = _N

_NTILE = 16
_EP = 24576
_ROWS_T = _NROW // _NTILE
_Z = _ROWS_T * _NCOL
_ZPAD = 28544


def _sc_scatter_body(edges_hbm, out_hbm, src_v, dst_v, bacc_v):
    c = lax.axis_index("c")
    s = lax.axis_index("s")
    w = c * 16 + s
    pltpu.sync_copy(edges_hbm.at[pl.ds(0, _EP)], src_v)
    pltpu.sync_copy(edges_hbm.at[pl.ds(_EP, _EP)], dst_v)

    ones16 = jnp.ones((16,), jnp.float32)
    zeros16 = jnp.zeros((16,), jnp.float32)

    @plsc.parallel_loop(0, _ZPAD // 16, unroll=8)
    def _zero_body(i):
        bacc_v[pl.ds(i * 16, 16)] = zeros16

    base = w * _ROWS_T

    @plsc.parallel_loop(0, _EP // 16, unroll=8)
    def _scatter_body(t):
        sv = src_v[pl.ds(t * 16, 16)]
        dv = dst_v[pl.ds(t * 16, 16)] - base
        loc = dv * _NCOL + sv
        valid = (loc >= 0) & (loc < _Z)
        plsc.addupdate_scatter(bacc_v, [loc], ones16, mask=valid)

    pltpu.sync_copy(bacc_v.at[pl.ds(0, _Z)], out_hbm.at[pl.ds(w * _Z, _Z)])


def _make_sc_scatter():
    return pl.kernel(
        _sc_scatter_body,
        out_type=jax.ShapeDtypeStruct((_NROW * _NCOL,), jnp.float32),
        mesh=plsc.VectorSubcoreMesh(core_axis_name="c", subcore_axis_name="s",
                                    num_cores=1),
        compiler_params=pltpu.CompilerParams(needs_layout_passes=False),
        scratch_types=[
            pltpu.VMEM((_EP,), jnp.int32),
            pltpu.VMEM((_EP,), jnp.int32),
            pltpu.VMEM((_ZPAD,), jnp.float32),
        ],
    )


def _tc_body(p_ref, x_ref, w1_ref, b1_ref, w2_ref, b2_ref, wlp_ref, blp_ref,
             out_ref):
    bmat = p_ref[pl.ds(0, _NP), :]
    rows = lax.broadcasted_iota(jnp.int32, (_NP, _NCOL), 0)
    cols = lax.broadcasted_iota(jnp.int32, (_NP, _NCOL), 1)
    eye = jnp.where((rows == cols) & (rows < _N), 1.0, 0.0)
    bmat = bmat + eye
    deg = jnp.sum(bmat, axis=1, keepdims=True)
    dinv = lax.rsqrt(jnp.maximum(deg, 1e-12))

    x = x_ref[...]
    h1 = jnp.dot(x, w1_ref[...], preferred_element_type=jnp.float32)
    agg1 = dinv * jnp.dot(bmat, dinv * h1,
                          preferred_element_type=jnp.float32) + b1_ref[...]
    res = jnp.dot(x, wlp_ref[...], preferred_element_type=jnp.float32)
    h = res + blp_ref[...] + agg1
    h = h * (1.0 / (1.0 + jnp.exp(-h)))

    rmask = lax.broadcasted_iota(jnp.int32, (_NP, 64), 0) < _N
    cnt = float(_N * 64)
    mu = jnp.sum(jnp.where(rmask, h, 0.0)) / cnt
    dev = jnp.where(rmask, h - mu, 0.0)
    var = jnp.sum(dev * dev) / cnt
    hn = (h - mu) * lax.rsqrt(var + 1e-5)

    h2 = jnp.dot(hn, w2_ref[...], preferred_element_type=jnp.float32)
    out_ref[...] = dinv * jnp.dot(bmat, dinv * h2,
                                  preferred_element_type=jnp.float32) + b2_ref[...]


def kernel(x, edge_index, W1, b1, W2, b2, Wlp, blp):
    ei = edge_index.astype(jnp.int32)
    pad = _EP - _E
    src_p = jnp.concatenate([ei[0], jnp.zeros((pad,), jnp.int32)])
    dst_p = jnp.concatenate([ei[1], jnp.full((pad,), _PAD_DST, jnp.int32)])
    edges = jnp.concatenate([src_p, dst_p])

    p = _make_sc_scatter()(edges).reshape(_NROW, _NCOL)

    xp = jnp.pad(x, ((0, _NP - _N), (0, 0)))
    out = pl.pallas_call(
        _tc_body,
        out_shape=jax.ShapeDtypeStruct((_NP, 64), jnp.float32),
    )(p, xp, W1, b1.reshape(1, -1), W2, b2.reshape(1, -1),
      Wlp, blp.reshape(1, -1))
    return out[:_N]


# speedup vs baseline: 22.1226x; 1.1328x over previous
import jax
import jax.numpy as jnp
from jax import lax
from jax.experimental import pallas as pl
from jax.experimental.pallas import tpu as pltpu
from jax.experimental.pallas import tpu_sc as plsc

_N = 646
_E = 20672
_NCOL = 648
_NP = 648
_NROW = 704

_NTILE = 16
_ROWS_T = _NROW // _NTILE
_Z = _ROWS_T * _NCOL
_ZPAD = 28544


def _sc_scatter_body(src_hbm, dst_hbm, out_hbm, src_v, dst_v, bacc_v):
    s = lax.axis_index("s")
    w = lax.axis_index("c") * 16 + s
    pltpu.sync_copy(src_hbm, src_v)
    pltpu.sync_copy(dst_hbm, dst_v)

    ones16 = jnp.ones((16,), jnp.float32)
    zeros16 = jnp.zeros((16,), jnp.float32)

    @plsc.parallel_loop(0, _ZPAD // 16, unroll=8)
    def _zero_body(i):
        bacc_v[pl.ds(i * 16, 16)] = zeros16

    base = w * _ROWS_T

    @plsc.parallel_loop(0, _E // 16, unroll=4)
    def _scatter_body(t):
        sv = src_v[pl.ds(t * 16, 16)]
        dv = dst_v[pl.ds(t * 16, 16)] - base
        loc = dv * _NCOL + sv
        valid = (loc >= 0) & (loc < _Z)
        plsc.addupdate_scatter(bacc_v, [loc], ones16, mask=valid)

    pltpu.sync_copy(bacc_v.at[pl.ds(0, _Z)], out_hbm.at[pl.ds(w * _Z, _Z)])


def _make_sc_scatter():
    return pl.kernel(
        _sc_scatter_body,
        out_type=jax.ShapeDtypeStruct((_NROW * _NCOL,), jnp.float32),
        mesh=plsc.VectorSubcoreMesh(core_axis_name="c", subcore_axis_name="s",
                                    num_cores=1),
        compiler_params=pltpu.CompilerParams(needs_layout_passes=False),
        scratch_types=[
            pltpu.VMEM((_E,), jnp.int32),
            pltpu.VMEM((_E,), jnp.int32),
            pltpu.VMEM((_ZPAD,), jnp.float32),
        ],
    )


def _tc_pre_body(x_ref, w1_ref, wlp_ref, blp_ref, h1_ref, res_ref):
    x = x_ref[...]
    pad2 = jnp.zeros((_NP - _N, 64), jnp.float32)
    h1 = jnp.dot(x, w1_ref[...], preferred_element_type=jnp.float32)
    h1_ref[...] = jnp.concatenate([h1, pad2])
    res = jnp.dot(x, wlp_ref[...], preferred_element_type=jnp.float32)
    res_ref[...] = jnp.concatenate([res + blp_ref[...], pad2])


def _tc_main_body(p_ref, h1_ref, res_ref, b1_ref, w2_ref, b2_ref, out_ref):
    bmat = p_ref[pl.ds(0, _NP), :]
    rows = lax.broadcasted_iota(jnp.int32, (_NP, _NCOL), 0)
    cols = lax.broadcasted_iota(jnp.int32, (_NP, _NCOL), 1)
    eye = jnp.where((rows == cols) & (rows < _N), 1.0, 0.0)
    bmat = bmat + eye
    deg = jnp.sum(bmat, axis=1, keepdims=True)
    dinv = lax.rsqrt(jnp.maximum(deg, 1e-12))

    agg1 = dinv * jnp.dot(bmat, dinv * h1_ref[...],
                          preferred_element_type=jnp.float32) + b1_ref[...]
    h = res_ref[...] + agg1
    h = h * (1.0 / (1.0 + jnp.exp(-h)))

    rmask = lax.broadcasted_iota(jnp.int32, (_NP, 64), 0) < _N
    cnt = float(_N * 64)
    mu = jnp.sum(jnp.where(rmask, h, 0.0)) / cnt
    dev = jnp.where(rmask, h - mu, 0.0)
    var = jnp.sum(dev * dev) / cnt
    hn = (h - mu) * lax.rsqrt(var + 1e-5)

    h2 = jnp.dot(hn, w2_ref[...], preferred_element_type=jnp.float32)
    out_ref[...] = dinv * jnp.dot(bmat, dinv * h2,
                                  preferred_element_type=jnp.float32) + b2_ref[...]


def kernel(x, edge_index, W1, b1, W2, b2, Wlp, blp):
    ei = edge_index.astype(jnp.int32)

    p = _make_sc_scatter()(ei[0], ei[1]).reshape(_NROW, _NCOL)

    h1p, resp = pl.pallas_call(
        _tc_pre_body,
        out_shape=(jax.ShapeDtypeStruct((_NP, 64), jnp.float32),
                   jax.ShapeDtypeStruct((_NP, 64), jnp.float32)),
    )(x, W1, Wlp, blp.reshape(1, -1))

    out = pl.pallas_call(
        _tc_main_body,
        out_shape=jax.ShapeDtypeStruct((_NP, 64), jnp.float32),
    )(p, h1p, resp, b1.reshape(1, -1), W2, b2.reshape(1, -1))
    return out[:_N]


# speedup vs baseline: 24.5636x; 1.1103x over previous
import jax
import jax.numpy as jnp
from jax import lax
from jax.experimental import pallas as pl
from jax.experimental.pallas import tpu as pltpu
from jax.experimental.pallas import tpu_sc as plsc

_N = 646
_E = 20672
_NCOL = 648
_NP = 648
_NROW = 704

_NTILE = 16
_ROWS_T = _NROW // _NTILE
_Z = _ROWS_T * _NCOL
_ZPAD = 28544


def _sc_scatter_body(flat_hbm, out_hbm, flat_v, bacc_v, sem):
    s = lax.axis_index("s")
    w = lax.axis_index("c") * 16 + s
    load = pltpu.async_copy(flat_hbm, flat_v, sem)

    zeros16 = jnp.zeros((16,), jnp.float32)

    @plsc.parallel_loop(0, _ZPAD // 16, unroll=8)
    def _zero_body(i):
        bacc_v[pl.ds(i * 16, 16)] = zeros16

    load.wait()

    ones16 = jnp.ones((16,), jnp.float32)
    base = jnp.int32(w * _Z)
    zbound = jnp.uint32(_Z)

    @plsc.parallel_loop(0, _E // 16, unroll=4)
    def _scatter_body(t):
        loc = flat_v[pl.ds(t * 16, 16)] - base
        valid = plsc.bitcast(loc, jnp.uint32) < zbound
        plsc.addupdate_scatter(bacc_v, [loc], ones16, mask=valid)

    pltpu.sync_copy(bacc_v.at[pl.ds(0, _Z)], out_hbm.at[pl.ds(w * _Z, _Z)])


def _make_sc_scatter():
    return pl.kernel(
        _sc_scatter_body,
        out_type=jax.ShapeDtypeStruct((_NROW * _NCOL,), jnp.float32),
        mesh=plsc.VectorSubcoreMesh(core_axis_name="c", subcore_axis_name="s",
                                    num_cores=1),
        compiler_params=pltpu.CompilerParams(needs_layout_passes=False),
        scratch_types=[
            pltpu.VMEM((_E,), jnp.int32),
            pltpu.VMEM((_ZPAD,), jnp.float32),
            pltpu.SemaphoreType.DMA,
        ],
    )


def _tc_pre_body(x_ref, w1_ref, wlp_ref, blp_ref, h1_ref, res_ref):
    x = x_ref[...]
    pad2 = jnp.zeros((_NP - _N, 64), jnp.float32)
    h1 = jnp.dot(x, w1_ref[...], preferred_element_type=jnp.float32)
    h1_ref[...] = jnp.concatenate([h1, pad2])
    res = jnp.dot(x, wlp_ref[...], preferred_element_type=jnp.float32)
    res_ref[...] = jnp.concatenate([res + blp_ref[...], pad2])


def _tc_main_body(p_ref, h1_ref, res_ref, b1_ref, w2_ref, b2_ref, out_ref):
    bmat = p_ref[pl.ds(0, _NP), :]
    rows = lax.broadcasted_iota(jnp.int32, (_NP, _NCOL), 0)
    cols = lax.broadcasted_iota(jnp.int32, (_NP, _NCOL), 1)
    eye = jnp.where((rows == cols) & (rows < _N), 1.0, 0.0)
    bmat = bmat + eye
    deg = jnp.sum(bmat, axis=1, keepdims=True)
    dinv = lax.rsqrt(jnp.maximum(deg, 1e-12))

    agg1 = dinv * jnp.dot(bmat, dinv * h1_ref[...],
                          preferred_element_type=jnp.float32) + b1_ref[...]
    h = res_ref[...] + agg1
    h = h * (1.0 / (1.0 + jnp.exp(-h)))

    rmask = lax.broadcasted_iota(jnp.int32, (_NP, 64), 0) < _N
    cnt = float(_N * 64)
    mu = jnp.sum(jnp.where(rmask, h, 0.0)) / cnt
    dev = jnp.where(rmask, h - mu, 0.0)
    var = jnp.sum(dev * dev) / cnt
    hn = (h - mu) * lax.rsqrt(var + 1e-5)

    h2 = jnp.dot(hn, w2_ref[...], preferred_element_type=jnp.float32)
    out_ref[...] = dinv * jnp.dot(bmat, dinv * h2,
                                  preferred_element_type=jnp.float32) + b2_ref[...]


def kernel(x, edge_index, W1, b1, W2, b2, Wlp, blp):
    ei = edge_index.astype(jnp.int32)
    flat = ei[1] * _NCOL + ei[0]

    p = _make_sc_scatter()(flat).reshape(_NROW, _NCOL)

    h1p, resp = pl.pallas_call(
        _tc_pre_body,
        out_shape=(jax.ShapeDtypeStruct((_NP, 64), jnp.float32),
                   jax.ShapeDtypeStruct((_NP, 64), jnp.float32)),
    )(x, W1, Wlp, blp.reshape(1, -1))

    out = pl.pallas_call(
        _tc_main_body,
        out_shape=jax.ShapeDtypeStruct((_NP, 64), jnp.float32),
    )(p, h1p, resp, b1.reshape(1, -1), W2, b2.reshape(1, -1))
    return out[:_N]
